# baseline (speedup 1.0000x reference)
import functools
import math

import jax
import jax.numpy as jnp
from jax import lax
from jax.experimental import pallas as pl
from jax.experimental.pallas import tpu as pltpu
from jax.experimental.pallas import tpu_sc as plsc

D_MODEL_ = 64
SCALE_ = math.sqrt(D_MODEL_)

_info = plsc.get_sparse_core_info()
_NC, _NS, _L = _info.num_cores, _info.num_subcores, _info.num_lanes
_NW = _NC * _NS

_CH = 128


def _make_kernel(B, V, D):
    assert B % (_NW * 2 * _CH) == 0
    b_per_w = B // _NW
    n_pairs = b_per_w // (2 * _CH)
    mesh = plsc.VectorSubcoreMesh(core_axis_name="c", subcore_axis_name="s")

    @functools.partial(
        pl.kernel,
        mesh=mesh,
        out_type=jax.ShapeDtypeStruct((B, D), jnp.float32),
        scratch_types=[
            pltpu.VMEM((b_per_w // _CH, _CH), jnp.int32),
            pltpu.VMEM((_CH, D), jnp.float32),
            pltpu.VMEM((_CH, D), jnp.float32),
            pltpu.VMEM((_CH, D), jnp.float32),
            pltpu.VMEM((_CH, D), jnp.float32),
            pltpu.SemaphoreType.DMA,
            pltpu.SemaphoreType.DMA,
            pltpu.SemaphoreType.DMA,
            pltpu.SemaphoreType.DMA,
        ],
        compiler_params=pltpu.CompilerParams(use_tc_tiling_on_sc=False),
    )
    def emb_kernel(idx_hbm, table_hbm, out_hbm, idx_v, g0, g1, w0, w1,
                   gs0, gs1, ws0, ws1):
        wid = lax.axis_index("s") * _NC + lax.axis_index("c")
        base = wid * b_per_w
        pltpu.sync_copy(idx_hbm.at[wid], idx_v)

        def gather(c, buf, sem):
            return pltpu.make_async_copy(
                table_hbm.at[idx_v.at[c]], buf, sem)

        def writeback(c, buf, sem):
            return pltpu.make_async_copy(
                buf, out_hbm.at[pl.ds(base + c * _CH, _CH)], sem)

        def scale(src, dst):
            def row(i, carry):
                for j in range(D // _L):
                    sl = pl.ds(j * _L, _L)
                    dst[i, sl] = src[i, sl] * SCALE_
                return carry
            lax.fori_loop(0, _CH, row, 0, unroll=2)

        gather(0, g0, gs0).start()
        gather(1, g1, gs1).start()

        def pair(t, carry):
            c0 = 2 * t
            c1 = c0 + 1

            gather(c0, g0, gs0).wait()

            @pl.when(t > 0)
            def _():
                writeback(c0, w0, ws0).wait()

            scale(g0, w0)

            @pl.when(t < n_pairs - 1)
            def _():
                gather(c0 + 2, g0, gs0).start()

            writeback(c0, w0, ws0).start()

            gather(c1, g1, gs1).wait()

            @pl.when(t > 0)
            def _():
                writeback(c1, w1, ws1).wait()

            scale(g1, w1)

            @pl.when(t < n_pairs - 1)
            def _():
                gather(c1 + 2, g1, gs1).start()

            writeback(c1, w1, ws1).start()
            return carry

        lax.fori_loop(0, n_pairs, pair, 0)
        writeback(2 * n_pairs - 2, w0, ws0).wait()
        writeback(2 * n_pairs - 1, w1, ws1).wait()

    return emb_kernel


def kernel(x, table):
    B = x.size
    V, D = table.shape
    idx3d = x.reshape(_NW, B // (_NW * _CH), _CH).astype(jnp.int32)
    out = _make_kernel(B, V, D)(idx3d, table)
    return out.reshape(x.shape + (D,))


# speedup vs baseline: 1.1061x; 1.1061x over previous
import functools
import math

import jax
import jax.numpy as jnp
from jax import lax
from jax.experimental import pallas as pl
from jax.experimental.pallas import tpu as pltpu
from jax.experimental.pallas import tpu_sc as plsc

D_MODEL_ = 64
SCALE_ = math.sqrt(D_MODEL_)

_info = plsc.get_sparse_core_info()
_NC, _NS, _L = _info.num_cores, _info.num_subcores, _info.num_lanes
_NW = _NC * _NS

_G = 8


def _make_kernel(B1, S, V, D):
    assert B1 % (_NW * 2 * _G) == 0
    seq_per_w = B1 // _NW
    n_pairs = seq_per_w // (2 * _G)
    mesh = plsc.VectorSubcoreMesh(core_axis_name="c", subcore_axis_name="s")

    @functools.partial(
        pl.kernel,
        mesh=mesh,
        out_type=jax.ShapeDtypeStruct((B1, S, D), jnp.float32),
        scratch_types=[
            pltpu.VMEM((seq_per_w, S), jnp.int32),
            pltpu.VMEM((_G, S, D), jnp.float32),
            pltpu.VMEM((_G, S, D), jnp.float32),
            pltpu.VMEM((_G, S, D), jnp.float32),
            pltpu.VMEM((_G, S, D), jnp.float32),
            pltpu.SemaphoreType.DMA,
            pltpu.SemaphoreType.DMA,
            pltpu.SemaphoreType.DMA,
            pltpu.SemaphoreType.DMA,
        ],
        compiler_params=pltpu.CompilerParams(use_tc_tiling_on_sc=False),
    )
    def emb_kernel(idx_hbm, table_hbm, out_hbm, idx_v, g0, g1, w0, w1,
                   gs0, gs1, ws0, ws1):
        wid = lax.axis_index("s") * _NC + lax.axis_index("c")
        base = wid * seq_per_w
        pltpu.sync_copy(idx_hbm.at[pl.ds(base, seq_per_w)], idx_v)

        def gather_start(c, buf, sem):
            for g in range(_G):
                pltpu.make_async_copy(
                    table_hbm.at[idx_v.at[c * _G + g]], buf.at[g], sem
                ).start()

        def gather_wait(c, buf, sem):
            for g in range(_G):
                pltpu.make_async_copy(
                    table_hbm.at[idx_v.at[c * _G + g]], buf.at[g], sem
                ).wait()

        def writeback(c, buf, sem):
            return pltpu.make_async_copy(
                buf, out_hbm.at[pl.ds(base + c * _G, _G)], sem)

        def scale(src, dst):
            def row(i, carry):
                for g in range(_G):
                    for j in range(D // _L):
                        sl = pl.ds(j * _L, _L)
                        dst[g, i, sl] = src[g, i, sl] * SCALE_
                return carry
            lax.fori_loop(0, S, row, 0)

        gather_start(0, g0, gs0)
        gather_start(1, g1, gs1)

        def pair(t, carry):
            c0 = 2 * t
            c1 = c0 + 1

            gather_wait(c0, g0, gs0)

            @pl.when(t > 0)
            def _():
                writeback(c0, w0, ws0).wait()

            scale(g0, w0)

            @pl.when(t < n_pairs - 1)
            def _():
                gather_start(c0 + 2, g0, gs0)

            writeback(c0, w0, ws0).start()

            gather_wait(c1, g1, gs1)

            @pl.when(t > 0)
            def _():
                writeback(c1, w1, ws1).wait()

            scale(g1, w1)

            @pl.when(t < n_pairs - 1)
            def _():
                gather_start(c1 + 2, g1, gs1)

            writeback(c1, w1, ws1).start()
            return carry

        lax.fori_loop(0, n_pairs, pair, 0)
        writeback(2 * n_pairs - 2, w0, ws0).wait()
        writeback(2 * n_pairs - 1, w1, ws1).wait()

    return emb_kernel


def kernel(x, table):
    B1, S = x.shape
    V, D = table.shape
    out = _make_kernel(B1, S, V, D)(x.astype(jnp.int32), table)
    return out
